# baseline (speedup 1.0000x reference)
import functools
import jax
import jax.numpy as jnp
from jax import lax
from jax.experimental import pallas as pl
from jax.experimental.pallas import tpu as pltpu
from jax.experimental.pallas import tpu_sc as plsc

NC, NS, L = 2, 16, 16
NW = NC * NS
D = 64
BATCH, SEQ = 4096, 200
SP = SEQ // 2
BPW = BATCH // NW
NBUF = 2
PPB = 10

_mesh = plsc.VectorSubcoreMesh(core_axis_name="c", subcore_axis_name="s")


@functools.partial(
    pl.kernel,
    out_type=jax.ShapeDtypeStruct((SP, BATCH, 2 * D), jnp.float32),
    mesh=_mesh,
    scratch_types=[
        pltpu.VMEM((NBUF, 2, BPW), jnp.int32),
        pltpu.VMEM((SEQ, D), jnp.float32),
        pltpu.VMEM((NBUF, 2, BPW, D), jnp.float32),
        pltpu.VMEM((NBUF, BPW, 2 * D), jnp.float32),
        pltpu.SemaphoreType.DMA,
        pltpu.SemaphoreType.DMA,
        pltpu.SemaphoreType.DMA,
    ],
    compiler_params=pltpu.CompilerParams(use_tc_tiling_on_sc=False),
)
def _emb_kernel(tok_hbm, table_hbm, pe_hbm, out_hbm,
                idx_v, pe_v, rows_v, out_v, gsem, ssem, isem):
    wid = lax.axis_index("s") * NC + lax.axis_index("c")
    b0 = wid * BPW
    pltpu.sync_copy(pe_hbm.at[pl.ds(0, SEQ)], pe_v)

    def fire_gathers(bi):
        for h in range(2):
            pltpu.async_copy(table_hbm.at[idx_v.at[bi, h]],
                             rows_v.at[bi, h], gsem)

    def fire_store(p, bi):
        pltpu.async_copy(out_v.at[bi], out_hbm.at[p, pl.ds(b0, BPW)], ssem)

    for bi in range(NBUF):
        pltpu.sync_copy(tok_hbm.at[pl.ds(2 * bi, 2), pl.ds(b0, BPW)],
                        idx_v.at[bi])
        fire_gathers(bi)

    def outer(t, _):
        for bi in range(NBUF):
            p = t * NBUF + bi
            with jax.named_scope("drain_gather"):
                for h in range(2):
                    pltpu.make_async_copy(
                        table_hbm.at[pl.ds(0, BPW)], rows_v.at[bi, h],
                        gsem).wait()

            @pl.when(p + NBUF < SP)
            def _():
                pltpu.async_copy(
                    tok_hbm.at[pl.ds(2 * (p + NBUF), 2), pl.ds(b0, BPW)],
                    idx_v.at[bi], isem)

            with jax.named_scope("wait_store"):
                @pl.when(p >= NBUF)
                def _():
                    pltpu.make_async_copy(
                        out_v.at[bi], out_hbm.at[0, pl.ds(b0, BPW)],
                        ssem).wait()

            ob = out_v.at[bi]
            pv = [pe_v[2 * p + h, pl.ds(k * L, L)]
                  for h in range(2) for k in range(D // L)]

            def row(bb, _):
                for h in range(2):
                    rh = rows_v.at[bi, h]
                    for k in range(D // L):
                        ob[bb, pl.ds(h * D + k * L, L)] = (
                            rh[bb, pl.ds(k * L, L)] * 8.0
                            + pv[h * (D // L) + k])
                return 0
            with jax.named_scope("pack"):
                lax.fori_loop(0, BPW, row, 0)

            with jax.named_scope("fire_store"):
                fire_store(p, bi)

            @pl.when(p + NBUF < SP)
            def _():
                pltpu.make_async_copy(
                    tok_hbm.at[pl.ds(0, 2), pl.ds(b0, BPW)], idx_v.at[bi],
                    isem).wait()
                fire_gathers(bi)
        return 0

    lax.fori_loop(0, SP // NBUF, outer, 0)

    for bi in range(NBUF):
        pltpu.make_async_copy(
            out_v.at[bi], out_hbm.at[0, pl.ds(b0, BPW)], ssem).wait()


def _tc_block(x_ref, o_ref):
    eye = jnp.eye(BPW, dtype=jnp.float32)
    for p in range(PPB):
        xp = x_ref[p]
        yp = lax.dot_general(xp, eye, (((0,), (0,)), ((), ())),
                             preferred_element_type=jnp.float32)
        o_ref[pl.ds(2 * p, 2)] = yp.reshape(2, 8, 1, 8, BPW)


_tc_transpose = pl.pallas_call(
    _tc_block,
    grid=(SP // PPB, NW),
    in_specs=[pl.BlockSpec((PPB, BPW, 2 * D), lambda j, w: (j, w, 0))],
    out_specs=pl.BlockSpec((2 * PPB, 8, 1, 8, BPW),
                           lambda j, w: (j, 0, w, 0, 0)),
    out_shape=jax.ShapeDtypeStruct((SEQ, 8, NW, 8, BPW), jnp.float32),
)


def kernel(token_sequences, embedding_weight, positional_embedding):
    tok_t = token_sequences.T
    pe = positional_embedding.reshape(positional_embedding.shape[1], D)
    packed = _emb_kernel(tok_t, embedding_weight, pe)
    out5 = _tc_transpose(packed)
    return out5.transpose(2, 4, 0, 1, 3).reshape(BATCH, SEQ, D)


# speedup vs baseline: 1.3168x; 1.3168x over previous
import functools
import jax
import jax.numpy as jnp
from jax import lax
from jax.experimental import pallas as pl
from jax.experimental.pallas import tpu as pltpu
from jax.experimental.pallas import tpu_sc as plsc

NC, NS, L = 2, 16, 16
NW = NC * NS
D = 64
BATCH, SEQ = 4096, 200
TOTAL = BATCH * SEQ
RPW = TOTAL // NW
C = SEQ
K = 2
CK = C // K
G = RPW // C
NBUF = 2

_mesh = plsc.VectorSubcoreMesh(core_axis_name="c", subcore_axis_name="s")


@functools.partial(
    pl.kernel,
    out_type=jax.ShapeDtypeStruct((TOTAL, D), jnp.float32),
    mesh=_mesh,
    scratch_types=[
        pltpu.VMEM((NBUF, K, CK), jnp.int32),
        pltpu.VMEM((NBUF, C, D), jnp.float32),
        pltpu.VMEM((NBUF, C, D), jnp.float32),
        pltpu.VMEM((C, D), jnp.float32),
        pltpu.SemaphoreType.DMA,
        pltpu.SemaphoreType.DMA,
        pltpu.SemaphoreType.DMA,
    ],
    compiler_params=pltpu.CompilerParams(use_tc_tiling_on_sc=False),
)
def _emb_kernel(tok_hbm, table_hbm, pe_hbm, out_hbm,
                idx_v, rows_v, out_v, pe_v, gsem, ssem, isem):
    wid = lax.axis_index("s") * NC + lax.axis_index("c")
    base = wid * RPW
    pltpu.sync_copy(pe_hbm.at[pl.ds(0, C)], pe_v)

    def fire_gathers(g, b):
        for j in range(K):
            pltpu.async_copy(table_hbm.at[idx_v.at[b, j]],
                             rows_v.at[b, pl.ds(j * CK, CK)], gsem)

    for b in range(NBUF):
        pltpu.sync_copy(tok_hbm.at[wid, pl.ds(b * K, K)], idx_v.at[b])
        fire_gathers(b, b)

    def outer(t, _):
        for b in range(NBUF):
            g = t * NBUF + b
            pltpu.make_async_copy(
                table_hbm.at[pl.ds(0, C)], rows_v.at[b], gsem).wait()

            @pl.when(g + NBUF < G)
            def _():
                pltpu.async_copy(
                    tok_hbm.at[wid, pl.ds((g + NBUF) * K, K)],
                    idx_v.at[b], isem)

            @pl.when(g >= NBUF)
            def _():
                pltpu.make_async_copy(
                    out_v.at[b], out_hbm.at[pl.ds(base, C)], ssem).wait()

            rb, ob = rows_v.at[b], out_v.at[b]

            def row(i, _):
                for v in range(D // L):
                    sl = pl.ds(v * L, L)
                    ob[i, sl] = rb[i, sl] * 8.0 + pe_v[i, sl]
                return 0
            lax.fori_loop(0, C, row, 0)

            pltpu.async_copy(out_v.at[b],
                             out_hbm.at[pl.ds(base + g * C, C)], ssem)

            @pl.when(g + NBUF < G)
            def _():
                pltpu.make_async_copy(
                    tok_hbm.at[wid, pl.ds(0, K)], idx_v.at[b], isem).wait()
                fire_gathers(g + NBUF, b)
        return 0

    lax.fori_loop(0, G // NBUF, outer, 0)

    for b in range(NBUF):
        pltpu.make_async_copy(
            out_v.at[b], out_hbm.at[pl.ds(base, C)], ssem).wait()


def kernel(token_sequences, embedding_weight, positional_embedding):
    tok = token_sequences.reshape(NW, RPW // CK, CK)
    pe = positional_embedding.reshape(positional_embedding.shape[1], D)
    out = _emb_kernel(tok, embedding_weight, pe)
    return out.reshape(BATCH, SEQ, D)
